# baseline (speedup 1.0000x reference)
import functools

import jax
import jax.numpy as jnp
from jax import lax
from jax.experimental import pallas as pl
from jax.experimental.pallas import tpu as pltpu
from jax.experimental.pallas import tpu_sc as plsc

N_EXPERTS = 64
D_MODEL = 4096
SEQ = 8192
BATCH = 4

S_SC = 2048
S_TC = SEQ - S_SC

CHUNK = 1024
N_CHUNKS = S_TC // CHUNK

SC_NC = 2
SC_NS = 16
SC_TILES = SC_NC * SC_NS
D_TILE = D_MODEL // SC_TILES
SC_TCHUNK = 256
SC_NCHUNKS = S_SC // SC_TCHUNK
LANES = 16
VECS = D_TILE // LANES


def _tc_reduce_body(x_ref, out_ref, acc_ref):
    c = pl.program_id(1)

    @pl.when(c == 0)
    def _init():
        acc_ref[...] = jnp.zeros_like(acc_ref)

    acc_ref[...] += jnp.sum(
        x_ref[0].reshape(CHUNK // 8, 8, D_MODEL), axis=0)

    @pl.when(c == N_CHUNKS - 1)
    def _store():
        out_ref[0] = acc_ref[...]


def _sc_reduce_kernel(x_hbm, out_hbm, buf0, buf1, accv, sem0, sem1):
    wid = lax.axis_index("s") * SC_NC + lax.axis_index("c")
    d0 = wid * D_TILE
    bufs = (buf0, buf1)
    sems = (sem0, sem1)
    for b in range(BATCH):
        copies = [None] * SC_NCHUNKS
        copies[0] = pltpu.async_copy(
            x_hbm.at[b, pl.ds(S_TC, SC_TCHUNK), pl.ds(d0, D_TILE)],
            bufs[0], sems[0])
        accs = tuple(jnp.zeros((LANES,), jnp.float32) for _ in range(VECS))
        for ci in range(SC_NCHUNKS):
            if ci + 1 < SC_NCHUNKS:
                copies[ci + 1] = pltpu.async_copy(
                    x_hbm.at[b,
                             pl.ds(S_TC + (ci + 1) * SC_TCHUNK, SC_TCHUNK),
                             pl.ds(d0, D_TILE)],
                    bufs[(ci + 1) % 2], sems[(ci + 1) % 2])
            copies[ci].wait()
            buf = bufs[ci % 2]

            def body(t, a, buf=buf):
                t0 = t * 4
                for dt in range(4):
                    a = tuple(
                        a[j] + buf[t0 + dt, pl.ds(j * LANES, LANES)]
                        for j in range(VECS))
                return a

            accs = lax.fori_loop(0, SC_TCHUNK // 4, body, accs)
        for j in range(VECS):
            accv[pl.ds(j * LANES, LANES)] = accs[j]
        pltpu.sync_copy(accv, out_hbm.at[b, pl.ds(d0, D_TILE)])


_sc_reduce = functools.partial(
    pl.kernel,
    mesh=plsc.VectorSubcoreMesh(core_axis_name="c", subcore_axis_name="s"),
    out_type=jax.ShapeDtypeStruct((BATCH, D_MODEL), jnp.float32),
    scratch_types=[
        pltpu.VMEM((SC_TCHUNK, D_TILE), jnp.float32),
        pltpu.VMEM((SC_TCHUNK, D_TILE), jnp.float32),
        pltpu.VMEM((D_TILE,), jnp.float32),
        pltpu.SemaphoreType.DMA,
        pltpu.SemaphoreType.DMA,
    ],
)(_sc_reduce_kernel)


def _finalize_body(tc_ref, sc_ref, w_ref, ph_ref, ts_ref, ti_ref, coh_ref):
    pooled = (jnp.sum(tc_ref[...], axis=1) + sc_ref[...]) * (1.0 / SEQ)
    amp = lax.dot_general(
        pooled, w_ref[...], (((1,), (1,)), ((), ())),
        preferred_element_type=jnp.float32,
    )
    ph = ph_ref[...]
    coh = jnp.abs(amp * (jnp.cos(ph) + jnp.sin(ph)))
    coh_ref[...] = coh

    iota = lax.broadcasted_iota(jnp.int32, (BATCH, N_EXPERTS), 1)
    m1 = jnp.max(coh, axis=1, keepdims=True)
    i1 = jnp.min(jnp.where(coh == m1, iota, N_EXPERTS), axis=1, keepdims=True)
    coh2 = jnp.where(iota == i1, -1.0, coh)
    m2 = jnp.max(coh2, axis=1, keepdims=True)
    i2 = jnp.min(jnp.where(coh2 == m2, iota, N_EXPERTS), axis=1, keepdims=True)
    ts_ref[...] = jnp.where(iota == 0, m1, jnp.where(iota == 1, m2, 0.0))
    ti_ref[...] = jnp.where(iota == 0, i1, jnp.where(iota == 1, i2, 0))


def kernel(x, W, phase_angles, top_k):
    sc_partial = _sc_reduce(x)
    tc_partial = pl.pallas_call(
        _tc_reduce_body,
        grid=(BATCH, N_CHUNKS),
        in_specs=[pl.BlockSpec((1, CHUNK, D_MODEL), lambda b, c: (b, c, 0))],
        out_specs=pl.BlockSpec((1, 8, D_MODEL), lambda b, c: (b, 0, 0)),
        out_shape=jax.ShapeDtypeStruct((BATCH, 8, D_MODEL), jnp.float32),
        scratch_shapes=[pltpu.VMEM((8, D_MODEL), jnp.float32)],
        compiler_params=pltpu.CompilerParams(
            dimension_semantics=("parallel", "arbitrary"),
        ),
    )(x)

    ph2 = phase_angles.reshape(1, N_EXPERTS)
    ts, ti, coherence = pl.pallas_call(
        _finalize_body,
        in_specs=[
            pl.BlockSpec(tc_partial.shape, lambda: (0, 0, 0)),
            pl.BlockSpec(sc_partial.shape, lambda: (0, 0)),
            pl.BlockSpec(W.shape, lambda: (0, 0)),
            pl.BlockSpec(ph2.shape, lambda: (0, 0)),
        ],
        out_specs=[
            pl.BlockSpec((BATCH, N_EXPERTS), lambda: (0, 0)),
            pl.BlockSpec((BATCH, N_EXPERTS), lambda: (0, 0)),
            pl.BlockSpec((BATCH, N_EXPERTS), lambda: (0, 0)),
        ],
        out_shape=[
            jax.ShapeDtypeStruct((BATCH, N_EXPERTS), jnp.float32),
            jax.ShapeDtypeStruct((BATCH, N_EXPERTS), jnp.int32),
            jax.ShapeDtypeStruct((BATCH, N_EXPERTS), jnp.float32),
        ],
    )(tc_partial, sc_partial, W, ph2)

    delta = (jnp.asarray(top_k, jnp.int32) - 2).astype(jnp.float32)
    return (ts[:, :2] + delta, ti[:, :2], coherence)


# speedup vs baseline: 1.0109x; 1.0109x over previous
import functools

import jax
import jax.numpy as jnp
from jax import lax
from jax.experimental import pallas as pl
from jax.experimental.pallas import tpu as pltpu
from jax.experimental.pallas import tpu_sc as plsc

N_EXPERTS = 64
D_MODEL = 4096
SEQ = 8192
BATCH = 4

S_SC = 1024
S_TC = SEQ - S_SC

CHUNK = 1024
N_CHUNKS = S_TC // CHUNK

SC_NC = 2
SC_NS = 16
SC_TILES = SC_NC * SC_NS
D_TILE = D_MODEL // SC_TILES
SC_TCHUNK = 256
SC_NCHUNKS = S_SC // SC_TCHUNK
LANES = 16
VECS = D_TILE // LANES


def _tc_reduce_body(x_ref, out_ref, acc_ref):
    c = pl.program_id(1)

    @pl.when(c == 0)
    def _init():
        acc_ref[...] = jnp.zeros_like(acc_ref)

    acc_ref[...] += jnp.sum(
        x_ref[0].reshape(CHUNK // 8, 8, D_MODEL), axis=0)

    @pl.when(c == N_CHUNKS - 1)
    def _store():
        out_ref[0] = acc_ref[...]


def _sc_reduce_kernel(x_hbm, out_hbm, buf0, buf1, accv, sem0, sem1):
    wid = lax.axis_index("s") * SC_NC + lax.axis_index("c")
    d0 = wid * D_TILE
    bufs = (buf0, buf1)
    sems = (sem0, sem1)
    for b in range(BATCH):
        copies = [None] * SC_NCHUNKS
        copies[0] = pltpu.async_copy(
            x_hbm.at[b, pl.ds(S_TC, SC_TCHUNK), pl.ds(d0, D_TILE)],
            bufs[0], sems[0])
        accs = tuple(jnp.zeros((LANES,), jnp.float32) for _ in range(VECS))
        for ci in range(SC_NCHUNKS):
            if ci + 1 < SC_NCHUNKS:
                copies[ci + 1] = pltpu.async_copy(
                    x_hbm.at[b,
                             pl.ds(S_TC + (ci + 1) * SC_TCHUNK, SC_TCHUNK),
                             pl.ds(d0, D_TILE)],
                    bufs[(ci + 1) % 2], sems[(ci + 1) % 2])
            copies[ci].wait()
            buf = bufs[ci % 2]

            def body(t, a, buf=buf):
                t0 = t * 8
                for dt in range(8):
                    a = tuple(
                        a[j] + buf[t0 + dt, pl.ds(j * LANES, LANES)]
                        for j in range(VECS))
                return a

            accs = lax.fori_loop(0, SC_TCHUNK // 8, body, accs)
        for j in range(VECS):
            accv[pl.ds(j * LANES, LANES)] = accs[j]
        pltpu.sync_copy(accv, out_hbm.at[b, pl.ds(d0, D_TILE)])


_sc_reduce = functools.partial(
    pl.kernel,
    mesh=plsc.VectorSubcoreMesh(core_axis_name="c", subcore_axis_name="s"),
    out_type=jax.ShapeDtypeStruct((BATCH, D_MODEL), jnp.float32),
    scratch_types=[
        pltpu.VMEM((SC_TCHUNK, D_TILE), jnp.float32),
        pltpu.VMEM((SC_TCHUNK, D_TILE), jnp.float32),
        pltpu.VMEM((D_TILE,), jnp.float32),
        pltpu.SemaphoreType.DMA,
        pltpu.SemaphoreType.DMA,
    ],
)(_sc_reduce_kernel)


def _finalize_body(tc_ref, sc_ref, w_ref, ph_ref, ts_ref, ti_ref, coh_ref):
    pooled = (jnp.sum(tc_ref[...], axis=1) + sc_ref[...]) * (1.0 / SEQ)
    amp = lax.dot_general(
        pooled, w_ref[...], (((1,), (1,)), ((), ())),
        preferred_element_type=jnp.float32,
    )
    ph = ph_ref[...]
    coh = jnp.abs(amp * (jnp.cos(ph) + jnp.sin(ph)))
    coh_ref[...] = coh

    iota = lax.broadcasted_iota(jnp.int32, (BATCH, N_EXPERTS), 1)
    m1 = jnp.max(coh, axis=1, keepdims=True)
    i1 = jnp.min(jnp.where(coh == m1, iota, N_EXPERTS), axis=1, keepdims=True)
    coh2 = jnp.where(iota == i1, -1.0, coh)
    m2 = jnp.max(coh2, axis=1, keepdims=True)
    i2 = jnp.min(jnp.where(coh2 == m2, iota, N_EXPERTS), axis=1, keepdims=True)
    ts_ref[...] = jnp.where(iota == 0, m1, jnp.where(iota == 1, m2, 0.0))
    ti_ref[...] = jnp.where(iota == 0, i1, jnp.where(iota == 1, i2, 0))


def kernel(x, W, phase_angles, top_k):
    sc_partial = _sc_reduce(x)
    tc_partial = pl.pallas_call(
        _tc_reduce_body,
        grid=(BATCH, N_CHUNKS),
        in_specs=[pl.BlockSpec((1, CHUNK, D_MODEL), lambda b, c: (b, c, 0))],
        out_specs=pl.BlockSpec((1, 8, D_MODEL), lambda b, c: (b, 0, 0)),
        out_shape=jax.ShapeDtypeStruct((BATCH, 8, D_MODEL), jnp.float32),
        scratch_shapes=[pltpu.VMEM((8, D_MODEL), jnp.float32)],
        compiler_params=pltpu.CompilerParams(
            dimension_semantics=("parallel", "arbitrary"),
        ),
    )(x)

    ph2 = phase_angles.reshape(1, N_EXPERTS)
    ts, ti, coherence = pl.pallas_call(
        _finalize_body,
        in_specs=[
            pl.BlockSpec(tc_partial.shape, lambda: (0, 0, 0)),
            pl.BlockSpec(sc_partial.shape, lambda: (0, 0)),
            pl.BlockSpec(W.shape, lambda: (0, 0)),
            pl.BlockSpec(ph2.shape, lambda: (0, 0)),
        ],
        out_specs=[
            pl.BlockSpec((BATCH, N_EXPERTS), lambda: (0, 0)),
            pl.BlockSpec((BATCH, N_EXPERTS), lambda: (0, 0)),
            pl.BlockSpec((BATCH, N_EXPERTS), lambda: (0, 0)),
        ],
        out_shape=[
            jax.ShapeDtypeStruct((BATCH, N_EXPERTS), jnp.float32),
            jax.ShapeDtypeStruct((BATCH, N_EXPERTS), jnp.int32),
            jax.ShapeDtypeStruct((BATCH, N_EXPERTS), jnp.float32),
        ],
    )(tc_partial, sc_partial, W, ph2)

    delta = (jnp.asarray(top_k, jnp.int32) - 2).astype(jnp.float32)
    return (ts[:, :2] + delta, ti[:, :2], coherence)


# speedup vs baseline: 1.0141x; 1.0031x over previous
import functools

import jax
import jax.numpy as jnp
from jax import lax
from jax.experimental import pallas as pl
from jax.experimental.pallas import tpu as pltpu
from jax.experimental.pallas import tpu_sc as plsc

N_EXPERTS = 64
D_MODEL = 4096
SEQ = 8192
BATCH = 4

S_SC = 512
S_TC = SEQ - S_SC

CHUNK = 512
N_CHUNKS = S_TC // CHUNK

SC_NC = 2
SC_NS = 16
SC_TILES = SC_NC * SC_NS
D_TILE = D_MODEL // SC_TILES
SC_TCHUNK = 256
SC_NCHUNKS = S_SC // SC_TCHUNK
LANES = 16
VECS = D_TILE // LANES


def _tc_reduce_body(x_ref, out_ref, acc_ref):
    c = pl.program_id(1)

    @pl.when(c == 0)
    def _init():
        acc_ref[...] = jnp.zeros_like(acc_ref)

    acc_ref[...] += jnp.sum(
        x_ref[0].reshape(CHUNK // 8, 8, D_MODEL), axis=0)

    @pl.when(c == N_CHUNKS - 1)
    def _store():
        out_ref[0] = acc_ref[...]


def _sc_reduce_kernel(x_hbm, out_hbm, buf0, buf1, accv, sem0, sem1):
    wid = lax.axis_index("s") * SC_NC + lax.axis_index("c")
    d0 = wid * D_TILE
    bufs = (buf0, buf1)
    sems = (sem0, sem1)
    for b in range(BATCH):
        copies = [None] * SC_NCHUNKS
        copies[0] = pltpu.async_copy(
            x_hbm.at[b, pl.ds(S_TC, SC_TCHUNK), pl.ds(d0, D_TILE)],
            bufs[0], sems[0])
        accs = tuple(jnp.zeros((LANES,), jnp.float32) for _ in range(VECS))
        for ci in range(SC_NCHUNKS):
            if ci + 1 < SC_NCHUNKS:
                copies[ci + 1] = pltpu.async_copy(
                    x_hbm.at[b,
                             pl.ds(S_TC + (ci + 1) * SC_TCHUNK, SC_TCHUNK),
                             pl.ds(d0, D_TILE)],
                    bufs[(ci + 1) % 2], sems[(ci + 1) % 2])
            copies[ci].wait()
            buf = bufs[ci % 2]

            def body(t, a, buf=buf):
                t0 = t * 8
                for dt in range(8):
                    a = tuple(
                        a[j] + buf[t0 + dt, pl.ds(j * LANES, LANES)]
                        for j in range(VECS))
                return a

            accs = lax.fori_loop(0, SC_TCHUNK // 8, body, accs)
        for j in range(VECS):
            accv[pl.ds(j * LANES, LANES)] = accs[j]
        pltpu.sync_copy(accv, out_hbm.at[b, pl.ds(d0, D_TILE)])


_sc_reduce = functools.partial(
    pl.kernel,
    mesh=plsc.VectorSubcoreMesh(core_axis_name="c", subcore_axis_name="s"),
    out_type=jax.ShapeDtypeStruct((BATCH, D_MODEL), jnp.float32),
    scratch_types=[
        pltpu.VMEM((SC_TCHUNK, D_TILE), jnp.float32),
        pltpu.VMEM((SC_TCHUNK, D_TILE), jnp.float32),
        pltpu.VMEM((D_TILE,), jnp.float32),
        pltpu.SemaphoreType.DMA,
        pltpu.SemaphoreType.DMA,
    ],
)(_sc_reduce_kernel)


def _finalize_body(tc_ref, sc_ref, w_ref, ph_ref, ts_ref, ti_ref, coh_ref):
    pooled = (jnp.sum(tc_ref[...], axis=1) + sc_ref[...]) * (1.0 / SEQ)
    amp = lax.dot_general(
        pooled, w_ref[...], (((1,), (1,)), ((), ())),
        preferred_element_type=jnp.float32,
    )
    ph = ph_ref[...]
    coh = jnp.abs(amp * (jnp.cos(ph) + jnp.sin(ph)))
    coh_ref[...] = coh

    iota = lax.broadcasted_iota(jnp.int32, (BATCH, N_EXPERTS), 1)
    m1 = jnp.max(coh, axis=1, keepdims=True)
    i1 = jnp.min(jnp.where(coh == m1, iota, N_EXPERTS), axis=1, keepdims=True)
    coh2 = jnp.where(iota == i1, -1.0, coh)
    m2 = jnp.max(coh2, axis=1, keepdims=True)
    i2 = jnp.min(jnp.where(coh2 == m2, iota, N_EXPERTS), axis=1, keepdims=True)
    ts_ref[...] = jnp.where(iota == 0, m1, jnp.where(iota == 1, m2, 0.0))
    ti_ref[...] = jnp.where(iota == 0, i1, jnp.where(iota == 1, i2, 0))


def kernel(x, W, phase_angles, top_k):
    sc_partial = _sc_reduce(x)
    tc_partial = pl.pallas_call(
        _tc_reduce_body,
        grid=(BATCH, N_CHUNKS),
        in_specs=[pl.BlockSpec((1, CHUNK, D_MODEL), lambda b, c: (b, c, 0))],
        out_specs=pl.BlockSpec((1, 8, D_MODEL), lambda b, c: (b, 0, 0)),
        out_shape=jax.ShapeDtypeStruct((BATCH, 8, D_MODEL), jnp.float32),
        scratch_shapes=[pltpu.VMEM((8, D_MODEL), jnp.float32)],
        compiler_params=pltpu.CompilerParams(
            dimension_semantics=("parallel", "arbitrary"),
        ),
    )(x)

    ph2 = phase_angles.reshape(1, N_EXPERTS)
    ts, ti, coherence = pl.pallas_call(
        _finalize_body,
        in_specs=[
            pl.BlockSpec(tc_partial.shape, lambda: (0, 0, 0)),
            pl.BlockSpec(sc_partial.shape, lambda: (0, 0)),
            pl.BlockSpec(W.shape, lambda: (0, 0)),
            pl.BlockSpec(ph2.shape, lambda: (0, 0)),
        ],
        out_specs=[
            pl.BlockSpec((BATCH, N_EXPERTS), lambda: (0, 0)),
            pl.BlockSpec((BATCH, N_EXPERTS), lambda: (0, 0)),
            pl.BlockSpec((BATCH, N_EXPERTS), lambda: (0, 0)),
        ],
        out_shape=[
            jax.ShapeDtypeStruct((BATCH, N_EXPERTS), jnp.float32),
            jax.ShapeDtypeStruct((BATCH, N_EXPERTS), jnp.int32),
            jax.ShapeDtypeStruct((BATCH, N_EXPERTS), jnp.float32),
        ],
    )(tc_partial, sc_partial, W, ph2)

    delta = (jnp.asarray(top_k, jnp.int32) - 2).astype(jnp.float32)
    return (ts[:, :2] + delta, ti[:, :2], coherence)


# speedup vs baseline: 1.1307x; 1.1150x over previous
import jax
import jax.numpy as jnp
from jax import lax
from jax.experimental import pallas as pl
from jax.experimental.pallas import tpu as pltpu

N_EXPERTS = 64
D_MODEL = 4096
SEQ = 8192
BATCH = 4
CHUNK = 512
N_CHUNKS = SEQ // CHUNK


def _router_body(x_ref, w_ref, ph_ref, ts_ref, ti_ref, coh_ref, acc_ref):
    c = pl.program_id(1)

    @pl.when(c == 0)
    def _init():
        acc_ref[...] = jnp.zeros_like(acc_ref)

    acc_ref[...] += jnp.sum(
        x_ref[0].reshape(CHUNK // 8, 8, D_MODEL), axis=0)

    @pl.when(c == N_CHUNKS - 1)
    def _finalize():
        pooled = jnp.sum(acc_ref[...], axis=0, keepdims=True) * (1.0 / SEQ)
        amp = lax.dot_general(
            pooled, w_ref[...], (((1,), (1,)), ((), ())),
            preferred_element_type=jnp.float32,
        )
        ph = ph_ref[...]
        coh = jnp.abs(amp * (jnp.cos(ph) + jnp.sin(ph)))
        coh_ref[0] = coh

        iota = lax.broadcasted_iota(jnp.int32, (1, N_EXPERTS), 1)
        m1 = jnp.max(coh, axis=1, keepdims=True)
        i1 = jnp.min(jnp.where(coh == m1, iota, N_EXPERTS),
                     axis=1, keepdims=True)
        coh2 = jnp.where(iota == i1, -1.0, coh)
        m2 = jnp.max(coh2, axis=1, keepdims=True)
        i2 = jnp.min(jnp.where(coh2 == m2, iota, N_EXPERTS),
                     axis=1, keepdims=True)
        ts_ref[0] = jnp.where(iota == 0, m1, jnp.where(iota == 1, m2, 0.0))
        ti_ref[0] = jnp.where(iota == 0, i1, jnp.where(iota == 1, i2, 0))


def kernel(x, W, phase_angles, top_k):
    ph2 = phase_angles.reshape(1, N_EXPERTS)
    ts_pad, ti_pad, coherence = pl.pallas_call(
        _router_body,
        grid=(BATCH, N_CHUNKS),
        in_specs=[
            pl.BlockSpec((1, CHUNK, D_MODEL), lambda b, c: (b, c, 0)),
            pl.BlockSpec((N_EXPERTS, D_MODEL), lambda b, c: (0, 0)),
            pl.BlockSpec((1, N_EXPERTS), lambda b, c: (0, 0)),
        ],
        out_specs=[
            pl.BlockSpec((1, 1, N_EXPERTS), lambda b, c: (b, 0, 0)),
            pl.BlockSpec((1, 1, N_EXPERTS), lambda b, c: (b, 0, 0)),
            pl.BlockSpec((1, 1, N_EXPERTS), lambda b, c: (b, 0, 0)),
        ],
        out_shape=[
            jax.ShapeDtypeStruct((BATCH, 1, N_EXPERTS), jnp.float32),
            jax.ShapeDtypeStruct((BATCH, 1, N_EXPERTS), jnp.int32),
            jax.ShapeDtypeStruct((BATCH, 1, N_EXPERTS), jnp.float32),
        ],
        scratch_shapes=[pltpu.VMEM((8, D_MODEL), jnp.float32)],
        compiler_params=pltpu.CompilerParams(
            dimension_semantics=("parallel", "arbitrary"),
        ),
    )(x, W, ph2)
    delta = (jnp.asarray(top_k, jnp.int32) - 2).astype(jnp.float32)
    top_scores = ts_pad[:, 0, :2] + delta
    top_idx = ti_pad[:, 0, :2]
    return (top_scores, top_idx, coherence[:, 0, :])
